# Initial kernel scaffold; baseline (speedup 1.0000x reference)
import jax
import jax.numpy as jnp
from jax.experimental import pallas as pl


def kernel(x, hs, cs, Wk, Wv, Wq, i2h, h2h, Ck, Cq, Cv, Co):
    raise NotImplementedError("write your pallas kernel here")


# speedup vs baseline: 1.4041x; 1.4041x over previous
import math

import jax
import jax.numpy as jnp
from jax.experimental import pallas as pl
from jax.experimental.pallas import tpu as pltpu

B = 64
D_IN = 1024
HID = 512
U = 8
K = 4
IKD = 64
IVD = 512
IH = 4
CKD = 64
CH = 4

def _bf(a):
    return a.astype(jnp.bfloat16)


def _dot(a, b):
    return jnp.dot(_bf(a), _bf(b), preferred_element_type=jnp.float32)


def _stage1_kernel(x_ref, hs_ref, Wk_ref, Wv_ref, Wq_ref, inp_ref, mask_ref):
    x = x_ref[...]
    kx = _dot(x, Wk_ref[...])
    Wv = Wv_ref[...]
    Wvm = (Wv[:, :IVD] + Wv[:, IVD:2 * IVD]
           + Wv[:, 2 * IVD:3 * IVD] + Wv[:, 3 * IVD:]) * 0.25
    v = _dot(x, Wvm)
    kxb = _bf(kx).astype(jnp.float32)
    s_cols = []
    for u in range(U):
        q_u = _dot(hs_ref[u], Wq_ref[u])
        qb = _bf(q_u).astype(jnp.float32)
        s_cols.append(jnp.sum(qb * kxb, axis=1, keepdims=True) * (1.0 / 32.0))
    s = jnp.concatenate(s_cols, axis=1)

    gt = s[:, None, :] > s[:, :, None]
    eq = s[:, None, :] == s[:, :, None]
    j_idx = jax.lax.broadcasted_iota(jnp.int32, (B, U, U), 2)
    i_idx = jax.lax.broadcasted_iota(jnp.int32, (B, U, U), 1)
    before = jnp.logical_or(gt, jnp.logical_and(eq, j_idx < i_idx))
    rank = jnp.sum(before.astype(jnp.float32), axis=2)
    mask = (rank < float(K)).astype(jnp.float32)

    a = jax.nn.sigmoid(s) * mask
    for u in range(U):
        inp_ref[u] = a[:, u:u + 1] * v
        mask_ref[u] = mask[:, u:u + 1]


def _stage2_kernel(inp_ref, hs_ref, cs_ref, mask_ref,
                   i2h_ref, h2h_ref, Ck_ref, Cq_ref, Cv_ref,
                   hbase_ref, csout_ref, keyc_ref, qryc_ref, valc_ref):
    inp = inp_ref[0]
    hs = hs_ref[0]
    cs = cs_ref[0]
    m = mask_ref[0]
    preact = _dot(inp, i2h_ref[0]) + _dot(hs, h2h_ref[0])
    i_t = jax.nn.sigmoid(preact[:, :HID])
    f_t = jax.nn.sigmoid(preact[:, HID:2 * HID])
    o_t = jax.nn.sigmoid(preact[:, 2 * HID:3 * HID])
    g_t = jnp.tanh(preact[:, 3 * HID:])
    c_t = cs * f_t + i_t * g_t
    h_t = o_t * jnp.tanh(c_t)
    csout_ref[0] = m * c_t + (1.0 - m) * cs
    hbase_ref[0] = m * h_t + (1.0 - m) * hs
    keyc_ref[0] = _dot(h_t, Ck_ref[0])
    qryc_ref[0] = _dot(h_t, Cq_ref[0])
    valc_ref[0] = _dot(h_t, Cv_ref[0])


def _stage3_kernel(qryc_ref, keyc_ref, valc_ref, mask_ref, hbase_ref, Co_ref,
                   hsout_ref):
    q = qryc_ref[0]
    m = mask_ref[0]
    keyc = keyc_ref[...]
    valc = valc_ref[...]
    inv = 1.0 / math.sqrt(CKD)
    ctx_parts = []
    for ch in range(CH):
        qh = q[:, ch * CKD:(ch + 1) * CKD]
        sc_cols = []
        for up in range(U):
            kh = keyc[up, :, ch * CKD:(ch + 1) * CKD]
            sc_cols.append(jnp.sum(qh * kh, axis=1, keepdims=True))
        sc = jnp.concatenate(sc_cols, axis=1) * inv
        sc = sc - jnp.max(sc, axis=1, keepdims=True)
        e = jnp.exp(sc)
        p = e / jnp.sum(e, axis=1, keepdims=True)
        ctx_h = p[:, 0:1] * valc[0, :, ch * HID:(ch + 1) * HID]
        for up in range(1, U):
            ctx_h = ctx_h + p[:, up:up + 1] * valc[up, :, ch * HID:(ch + 1) * HID]
        ctx_parts.append(ctx_h)
    ctx = jnp.concatenate(ctx_parts, axis=1)
    delta = _dot(ctx, Co_ref[0])
    hsout_ref[0] = hbase_ref[0] + m * delta


def kernel(x, hs, cs, Wk, Wv, Wq, i2h, h2h, Ck, Cq, Cv, Co):
    x2 = x[:, 0, :]
    hs_t = hs.transpose(1, 0, 2)
    cs_t = cs.transpose(1, 0, 2)

    inp_t, mask_t = pl.pallas_call(
        _stage1_kernel,
        out_shape=[
            jax.ShapeDtypeStruct((U, B, IVD), jnp.float32),
            jax.ShapeDtypeStruct((U, B, 1), jnp.float32),
        ],
    )(x2, hs_t, Wk, Wv, Wq)

    unit_block = lambda d: pl.BlockSpec((1, B, d), lambda u: (u, 0, 0))
    wblock = lambda d_in, d_out: pl.BlockSpec((1, d_in, d_out), lambda u: (u, 0, 0))

    hbase_t, csout_t, keyc, qryc, valc = pl.pallas_call(
        _stage2_kernel,
        grid=(U,),
        in_specs=[
            unit_block(IVD),
            unit_block(HID),
            unit_block(HID),
            unit_block(1),
            wblock(IVD, 4 * HID),
            wblock(HID, 4 * HID),
            wblock(HID, CH * CKD),
            wblock(HID, CH * CKD),
            wblock(HID, CH * HID),
        ],
        out_specs=[
            unit_block(HID),
            unit_block(HID),
            unit_block(CH * CKD),
            unit_block(CH * CKD),
            unit_block(CH * HID),
        ],
        out_shape=[
            jax.ShapeDtypeStruct((U, B, HID), jnp.float32),
            jax.ShapeDtypeStruct((U, B, HID), jnp.float32),
            jax.ShapeDtypeStruct((U, B, CH * CKD), jnp.float32),
            jax.ShapeDtypeStruct((U, B, CH * CKD), jnp.float32),
            jax.ShapeDtypeStruct((U, B, CH * HID), jnp.float32),
        ],
    )(inp_t, hs_t, cs_t, mask_t, i2h, h2h, Ck, Cq, Cv)

    full_block = lambda d: pl.BlockSpec((U, B, d), lambda u: (0, 0, 0))
    hsout_t = pl.pallas_call(
        _stage3_kernel,
        grid=(U,),
        in_specs=[
            unit_block(CH * CKD),
            full_block(CH * CKD),
            full_block(CH * HID),
            unit_block(1),
            unit_block(HID),
            wblock(CH * HID, HID),
        ],
        out_specs=unit_block(HID),
        out_shape=jax.ShapeDtypeStruct((U, B, HID), jnp.float32),
    )(qryc, keyc, valc, mask_t, hbase_t, Co)

    return hsout_t.transpose(1, 0, 2), csout_t.transpose(1, 0, 2)
